# baseline (device time: 68058 ns/iter reference)
import jax
import jax.numpy as jnp
from jax import lax
from jax.experimental import pallas as pl
from jax.experimental.pallas import tpu as pltpu

N_DEV = 8
M_BLK = 512
K_BLK = 512
N_TOT = 8192
N_TILES = 8
N_TILE = N_TOT // N_TILES
N_WBUF = 8
PREFETCH = 6

COMM_DTYPE = jnp.float8_e5m2


def kernel(x, w_mat, scale_x, scale_w):
    m_tot, k_shard = x.shape
    k_tot, n_tot = w_mat.shape
    assert m_tot == N_DEV * M_BLK and k_shard == K_BLK
    assert k_tot == N_DEV * K_BLK and n_tot == N_TOT

    def body(x_ref, w_ref, sx_ref, sw_ref, out_ref,
             sendbuf, comm, wbuf, obuf, xbuf, send_sems, recv_sems, local_sem,
             copy_sems, out_sems, x_sem):
        my = lax.axis_index("i")

        n_steps = N_DEV * N_TILES
        ksrcs = [lax.rem(my - d + N_DEV, N_DEV) for d in range(N_DEV)]

        def w_tile_copy(t):
            d, n = divmod(t, N_TILES)
            return pltpu.make_async_copy(
                w_ref.at[pl.ds(ksrcs[d] * K_BLK, K_BLK),
                         pl.ds(n * N_TILE, N_TILE)],
                wbuf.at[t % N_WBUF],
                copy_sems.at[t % N_WBUF],
            )

        x_cp = pltpu.make_async_copy(x_ref, xbuf, x_sem)
        x_cp.start()
        for t in range(PREFETCH):
            w_tile_copy(t).start()

        barrier = pltpu.get_barrier_semaphore()
        for p in range(1, N_DEV):
            pl.semaphore_signal(
                barrier, inc=1,
                device_id=(lax.rem(my + p, N_DEV),),
                device_id_type=pl.DeviceIdType.MESH,
            )
        pl.semaphore_wait(barrier, N_DEV - 1)

        x_cp.wait()
        for j in range(N_DEV):
            sendbuf[j, :, :] = xbuf[pl.ds(j * M_BLK, M_BLK), :].astype(COMM_DTYPE)

        local_cp = pltpu.make_async_copy(sendbuf.at[my], comm.at[0], local_sem)
        local_cp.start()

        rdmas = []
        for d in range(1, N_DEV):
            tgt = lax.rem(my + d, N_DEV)
            r = pltpu.make_async_remote_copy(
                src_ref=sendbuf.at[tgt],
                dst_ref=comm.at[d],
                send_sem=send_sems.at[d],
                recv_sem=recv_sems.at[d],
                device_id=(tgt,),
                device_id_type=pl.DeviceIdType.MESH,
            )
            r.start()
            rdmas.append(r)

        local_cp.wait()

        s = sx_ref[0] * sw_ref[0]
        a = comm[0, :, :]
        for t in range(n_steps):
            d, n = divmod(t, N_TILES)
            if t + PREFETCH < n_steps:
                w_tile_copy(t + PREFETCH).start()
            if n == 0 and d > 0:
                rdmas[d - 1].wait_recv()
                a = comm[d, :, :]
            w_tile_copy(t).wait()
            contrib = lax.dot_general(
                a, wbuf[t % N_WBUF, :, :].astype(COMM_DTYPE),
                (((1,), (0,)), ((), ())),
                preferred_element_type=jnp.float32,
            )
            sl = pl.ds(n * N_TILE, N_TILE)
            if d == 0:
                obuf[:, sl] = contrib
            elif d < N_DEV - 1:
                obuf[:, sl] = obuf[:, sl] + contrib
            else:
                y = (obuf[:, sl] + contrib) * s
                z = jnp.clip(y, -60.0, 60.0)
                obuf[:, sl] = y / (1.0 + jnp.exp(-z))
                pltpu.make_async_copy(
                    obuf.at[:, sl], out_ref.at[:, sl], out_sems.at[n]
                ).start()

        for n in range(N_TILES):
            sl = pl.ds(n * N_TILE, N_TILE)
            pltpu.make_async_copy(
                obuf.at[:, sl], out_ref.at[:, sl], out_sems.at[n]
            ).wait()

        for r in rdmas:
            r.wait_send()

    return pl.pallas_call(
        body,
        out_shape=jax.ShapeDtypeStruct((M_BLK, N_TOT), jnp.float32),
        in_specs=[
            pl.BlockSpec(memory_space=pl.ANY),
            pl.BlockSpec(memory_space=pl.ANY),
            pl.BlockSpec(memory_space=pltpu.SMEM),
            pl.BlockSpec(memory_space=pltpu.SMEM),
        ],
        out_specs=pl.BlockSpec(memory_space=pl.ANY),
        scratch_shapes=[
            pltpu.VMEM((N_DEV, M_BLK, K_BLK), COMM_DTYPE),
            pltpu.VMEM((N_DEV, M_BLK, K_BLK), COMM_DTYPE),
            pltpu.VMEM((N_WBUF, K_BLK, N_TILE), jnp.float32),
            pltpu.VMEM((M_BLK, N_TOT), jnp.float32),
            pltpu.VMEM((N_DEV * M_BLK, K_BLK), jnp.float32),
            pltpu.SemaphoreType.DMA((N_DEV,)),
            pltpu.SemaphoreType.DMA((N_DEV,)),
            pltpu.SemaphoreType.DMA,
            pltpu.SemaphoreType.DMA((N_WBUF,)),
            pltpu.SemaphoreType.DMA((N_TILES,)),
            pltpu.SemaphoreType.DMA,
        ],
        compiler_params=pltpu.CompilerParams(
            collective_id=0,
            vmem_limit_bytes=56 * 1024 * 1024,
        ),
    )(x, w_mat, scale_x, scale_w)


# device time: 67098 ns/iter; 1.0143x vs baseline; 1.0143x over previous
import jax
import jax.numpy as jnp
from jax import lax
from jax.experimental import pallas as pl
from jax.experimental.pallas import tpu as pltpu

N_DEV = 8
M_BLK = 512
K_BLK = 512
N_TOT = 8192
N_TILES = 2
N_TILE = N_TOT // N_TILES
N_WBUF = 3
PREFETCH = 2

COMM_DTYPE = jnp.float8_e5m2


def kernel(x, w_mat, scale_x, scale_w):
    m_tot, k_shard = x.shape
    k_tot, n_tot = w_mat.shape
    assert m_tot == N_DEV * M_BLK and k_shard == K_BLK
    assert k_tot == N_DEV * K_BLK and n_tot == N_TOT

    def body(x_ref, w_ref, sx_ref, sw_ref, out_ref,
             sendbuf, comm, wbuf, obuf, xbuf, send_sems, recv_sems, local_sem,
             copy_sems, out_sems, x_sem):
        my = lax.axis_index("i")

        n_steps = N_DEV * N_TILES
        ksrcs = [lax.rem(my - d + N_DEV, N_DEV) for d in range(N_DEV)]

        def w_tile_copy(t):
            d, n = divmod(t, N_TILES)
            return pltpu.make_async_copy(
                w_ref.at[pl.ds(ksrcs[d] * K_BLK, K_BLK),
                         pl.ds(n * N_TILE, N_TILE)],
                wbuf.at[t % N_WBUF],
                copy_sems.at[t % N_WBUF],
            )

        x_cp = pltpu.make_async_copy(x_ref, xbuf, x_sem)
        x_cp.start()
        for t in range(PREFETCH):
            w_tile_copy(t).start()

        barrier = pltpu.get_barrier_semaphore()
        for p in range(1, N_DEV):
            pl.semaphore_signal(
                barrier, inc=1,
                device_id=(lax.rem(my + p, N_DEV),),
                device_id_type=pl.DeviceIdType.MESH,
            )
        pl.semaphore_wait(barrier, N_DEV - 1)

        x_cp.wait()
        for j in range(N_DEV):
            sendbuf[j, :, :] = xbuf[pl.ds(j * M_BLK, M_BLK), :].astype(COMM_DTYPE)

        local_cp = pltpu.make_async_copy(sendbuf.at[my], comm.at[0], local_sem)
        local_cp.start()

        rdmas = []
        for d in range(1, N_DEV):
            tgt = lax.rem(my + d, N_DEV)
            r = pltpu.make_async_remote_copy(
                src_ref=sendbuf.at[tgt],
                dst_ref=comm.at[d],
                send_sem=send_sems.at[d],
                recv_sem=recv_sems.at[d],
                device_id=(tgt,),
                device_id_type=pl.DeviceIdType.MESH,
            )
            r.start()
            rdmas.append(r)

        local_cp.wait()

        s = sx_ref[0] * sw_ref[0]
        a = comm[0, :, :]
        for t in range(n_steps):
            d, n = divmod(t, N_TILES)
            if t + PREFETCH < n_steps:
                w_tile_copy(t + PREFETCH).start()
            if n == 0 and d > 0:
                rdmas[d - 1].wait_recv()
                a = comm[d, :, :]
            w_tile_copy(t).wait()
            contrib = lax.dot_general(
                a, wbuf[t % N_WBUF, :, :].astype(COMM_DTYPE),
                (((1,), (0,)), ((), ())),
                preferred_element_type=jnp.float32,
            )
            sl = pl.ds(n * N_TILE, N_TILE)
            if d == 0:
                obuf[:, sl] = contrib
            elif d < N_DEV - 1:
                obuf[:, sl] = obuf[:, sl] + contrib
            else:
                y = (obuf[:, sl] + contrib) * s
                z = jnp.clip(y, -60.0, 60.0)
                obuf[:, sl] = y / (1.0 + jnp.exp(-z))
                pltpu.make_async_copy(
                    obuf.at[:, sl], out_ref.at[:, sl], out_sems.at[n]
                ).start()

        for n in range(N_TILES):
            sl = pl.ds(n * N_TILE, N_TILE)
            pltpu.make_async_copy(
                obuf.at[:, sl], out_ref.at[:, sl], out_sems.at[n]
            ).wait()

        for r in rdmas:
            r.wait_send()

    return pl.pallas_call(
        body,
        out_shape=jax.ShapeDtypeStruct((M_BLK, N_TOT), jnp.float32),
        in_specs=[
            pl.BlockSpec(memory_space=pl.ANY),
            pl.BlockSpec(memory_space=pl.ANY),
            pl.BlockSpec(memory_space=pltpu.SMEM),
            pl.BlockSpec(memory_space=pltpu.SMEM),
        ],
        out_specs=pl.BlockSpec(memory_space=pl.ANY),
        scratch_shapes=[
            pltpu.VMEM((N_DEV, M_BLK, K_BLK), COMM_DTYPE),
            pltpu.VMEM((N_DEV, M_BLK, K_BLK), COMM_DTYPE),
            pltpu.VMEM((N_WBUF, K_BLK, N_TILE), jnp.float32),
            pltpu.VMEM((M_BLK, N_TOT), jnp.float32),
            pltpu.VMEM((N_DEV * M_BLK, K_BLK), jnp.float32),
            pltpu.SemaphoreType.DMA((N_DEV,)),
            pltpu.SemaphoreType.DMA((N_DEV,)),
            pltpu.SemaphoreType.DMA,
            pltpu.SemaphoreType.DMA((N_WBUF,)),
            pltpu.SemaphoreType.DMA((N_TILES,)),
            pltpu.SemaphoreType.DMA,
        ],
        compiler_params=pltpu.CompilerParams(
            collective_id=0,
            vmem_limit_bytes=56 * 1024 * 1024,
        ),
    )(x, w_mat, scale_x, scale_w)


# device time: 65233 ns/iter; 1.0433x vs baseline; 1.0286x over previous
import jax
import jax.numpy as jnp
from jax import lax
from jax.experimental import pallas as pl
from jax.experimental.pallas import tpu as pltpu

N_DEV = 8
M_BLK = 512
K_BLK = 512
N_TOT = 8192
N_TILES = 4
N_TILE = N_TOT // N_TILES
N_WBUF = 4
PREFETCH = 3

COMM_DTYPE = jnp.float8_e5m2


def kernel(x, w_mat, scale_x, scale_w):
    m_tot, k_shard = x.shape
    k_tot, n_tot = w_mat.shape
    assert m_tot == N_DEV * M_BLK and k_shard == K_BLK
    assert k_tot == N_DEV * K_BLK and n_tot == N_TOT

    def body(x_ref, w_ref, sx_ref, sw_ref, out_ref,
             sendbuf, comm, wbuf, obuf, xbuf, send_sems, recv_sems, local_sem,
             copy_sems, out_sems, x_sem):
        my = lax.axis_index("i")

        n_steps = N_DEV * N_TILES
        ksrcs = [lax.rem(my - d + N_DEV, N_DEV) for d in range(N_DEV)]

        def w_tile_copy(t):
            d, n = divmod(t, N_TILES)
            return pltpu.make_async_copy(
                w_ref.at[pl.ds(ksrcs[d] * K_BLK, K_BLK),
                         pl.ds(n * N_TILE, N_TILE)],
                wbuf.at[t % N_WBUF],
                copy_sems.at[t % N_WBUF],
            )

        x_cp = pltpu.make_async_copy(x_ref, xbuf, x_sem)
        x_cp.start()
        for t in range(PREFETCH):
            w_tile_copy(t).start()

        barrier = pltpu.get_barrier_semaphore()
        for p in range(1, N_DEV):
            pl.semaphore_signal(
                barrier, inc=1,
                device_id=(lax.rem(my + p, N_DEV),),
                device_id_type=pl.DeviceIdType.MESH,
            )
        pl.semaphore_wait(barrier, N_DEV - 1)

        x_cp.wait()
        for j in range(N_DEV):
            sendbuf[j, :, :] = xbuf[pl.ds(j * M_BLK, M_BLK), :].astype(COMM_DTYPE)

        local_cp = pltpu.make_async_copy(sendbuf.at[my], comm.at[0], local_sem)
        local_cp.start()

        rdmas = []
        for d in range(1, N_DEV):
            tgt = lax.rem(my + d, N_DEV)
            r = pltpu.make_async_remote_copy(
                src_ref=sendbuf.at[tgt],
                dst_ref=comm.at[d],
                send_sem=send_sems.at[d],
                recv_sem=recv_sems.at[d],
                device_id=(tgt,),
                device_id_type=pl.DeviceIdType.MESH,
            )
            r.start()
            rdmas.append(r)

        local_cp.wait()

        s = sx_ref[0] * sw_ref[0]
        a = comm[0, :, :]
        for t in range(n_steps):
            d, n = divmod(t, N_TILES)
            if t + PREFETCH < n_steps:
                w_tile_copy(t + PREFETCH).start()
            if n == 0 and d > 0:
                rdmas[d - 1].wait_recv()
                a = comm[d, :, :]
            w_tile_copy(t).wait()
            contrib = lax.dot_general(
                a, wbuf[t % N_WBUF, :, :].astype(COMM_DTYPE),
                (((1,), (0,)), ((), ())),
                preferred_element_type=jnp.float32,
            )
            sl = pl.ds(n * N_TILE, N_TILE)
            if d == 0:
                obuf[:, sl] = contrib
            elif d < N_DEV - 1:
                obuf[:, sl] = obuf[:, sl] + contrib
            else:
                y = (obuf[:, sl] + contrib) * s
                z = jnp.clip(y, -60.0, 60.0)
                obuf[:, sl] = y / (1.0 + jnp.exp(-z))
                pltpu.make_async_copy(
                    obuf.at[:, sl], out_ref.at[:, sl], out_sems.at[n]
                ).start()

        for n in range(N_TILES):
            sl = pl.ds(n * N_TILE, N_TILE)
            pltpu.make_async_copy(
                obuf.at[:, sl], out_ref.at[:, sl], out_sems.at[n]
            ).wait()

        for r in rdmas:
            r.wait_send()

    return pl.pallas_call(
        body,
        out_shape=jax.ShapeDtypeStruct((M_BLK, N_TOT), jnp.float32),
        in_specs=[
            pl.BlockSpec(memory_space=pl.ANY),
            pl.BlockSpec(memory_space=pl.ANY),
            pl.BlockSpec(memory_space=pltpu.SMEM),
            pl.BlockSpec(memory_space=pltpu.SMEM),
        ],
        out_specs=pl.BlockSpec(memory_space=pl.ANY),
        scratch_shapes=[
            pltpu.VMEM((N_DEV, M_BLK, K_BLK), COMM_DTYPE),
            pltpu.VMEM((N_DEV, M_BLK, K_BLK), COMM_DTYPE),
            pltpu.VMEM((N_WBUF, K_BLK, N_TILE), jnp.float32),
            pltpu.VMEM((M_BLK, N_TOT), jnp.float32),
            pltpu.VMEM((N_DEV * M_BLK, K_BLK), jnp.float32),
            pltpu.SemaphoreType.DMA((N_DEV,)),
            pltpu.SemaphoreType.DMA((N_DEV,)),
            pltpu.SemaphoreType.DMA,
            pltpu.SemaphoreType.DMA((N_WBUF,)),
            pltpu.SemaphoreType.DMA((N_TILES,)),
            pltpu.SemaphoreType.DMA,
        ],
        compiler_params=pltpu.CompilerParams(
            collective_id=0,
            vmem_limit_bytes=56 * 1024 * 1024,
        ),
    )(x, w_mat, scale_x, scale_w)
